# Initial kernel scaffold; baseline (speedup 1.0000x reference)
import jax
import jax.numpy as jnp
from jax.experimental import pallas as pl


def kernel(x, edge_index, W1_l, W1_r, b1, W2_l, W2_r, b2, Wc, bc):
    raise NotImplementedError("write your pallas kernel here")


# speedup vs baseline: 6.8649x; 6.8649x over previous
import functools

import jax
import jax.numpy as jnp
from jax import lax
from jax.experimental import pallas as pl
from jax.experimental.pallas import tpu as pltpu
from jax.experimental.pallas import tpu_sc as plsc

C = 80
NS = 16
NC = 2
_PREC = jax.lax.Precision.HIGHEST


def _sc_agg_body(NP, n_blocks, idxb, rows_out, table, srccat, dstcat, zrows,
                 out_agg, acc_sh, srcv, dstv, rows_a, rows_b, sem_a, sem_b):
    c = lax.axis_index("c")
    s = lax.axis_index("s")

    stripe = pl.ds(s * rows_out, rows_out)
    pltpu.sync_copy(zrows.at[stripe], acc_sh.at[stripe])
    plsc.subcore_barrier()

    for b in range(n_blocks):
        pltpu.sync_copy(srccat.at[c, s, b], srcv)
        pltpu.sync_copy(dstcat.at[c, s, b], dstv)

        pltpu.async_copy(table.at[srcv.at[0]], rows_a, sem_a)

        def body(i, carry):
            ja = 2 * i
            jb = 2 * i + 1
            pltpu.make_async_copy(table.at[srcv.at[ja]], rows_a, sem_a).wait()
            pltpu.async_copy(table.at[srcv.at[jb]], rows_b, sem_b)
            pltpu.sync_copy(rows_a, acc_sh.at[dstv.at[ja]], add=True)
            pltpu.make_async_copy(table.at[srcv.at[jb]], rows_b, sem_b).wait()

            @pl.when(jb + 1 < idxb)
            def _():
                pltpu.async_copy(table.at[srcv.at[ja + 2]], rows_a, sem_a)

            pltpu.sync_copy(rows_b, acc_sh.at[dstv.at[jb]], add=True)
            return carry

        lax.fori_loop(0, idxb // 2, body, 0)
        if idxb % 2:
            pltpu.make_async_copy(table.at[srcv.at[idxb - 1]], rows_a,
                                  sem_a).wait()
            pltpu.sync_copy(rows_a, acc_sh.at[dstv.at[idxb - 1]], add=True)
    plsc.subcore_barrier()

    pltpu.sync_copy(acc_sh.at[stripe], out_agg.at[c, stripe])


@jax.jit
def _sc_aggregate(table, srccat, dstcat, zrows):
    n_blocks, idxb = dstcat.shape[2], dstcat.shape[3]
    NP = zrows.shape[0]
    rows_out = NP // NS

    scratch = (
        pltpu.VMEM_SHARED((NP, 128), jnp.float32),
        pltpu.VMEM((idxb, C), jnp.int32),
        pltpu.VMEM((idxb, C), jnp.int32),
        pltpu.VMEM((C, 128), jnp.float32),
        pltpu.VMEM((C, 128), jnp.float32),
        pltpu.SemaphoreType.DMA,
        pltpu.SemaphoreType.DMA,
    )
    mesh = plsc.VectorSubcoreMesh(core_axis_name="c", subcore_axis_name="s")
    body = functools.partial(_sc_agg_body, NP, n_blocks, idxb, rows_out)
    fn = pl.kernel(body,
                   out_type=jax.ShapeDtypeStruct((NC, NP, 128), jnp.float32),
                   mesh=mesh, scratch_types=scratch)
    return fn(table, srccat, dstcat, zrows)


def _sc_deg_body(NP, n_blocks, idxb, rows_out, dstcat, zrows, ones_in,
                 out_deg, deg_sh, dstv, ones_v):
    c = lax.axis_index("c")
    s = lax.axis_index("s")

    stripe = pl.ds(s * rows_out, rows_out)
    pltpu.sync_copy(zrows.at[stripe], deg_sh.at[stripe])
    pltpu.sync_copy(ones_in, ones_v)
    plsc.subcore_barrier()

    for b in range(n_blocks):
        pltpu.sync_copy(dstcat.at[c, s, b], dstv)

        def body(j, carry):
            pltpu.sync_copy(ones_v, deg_sh.at[dstv.at[j]], add=True)
            return carry

        lax.fori_loop(0, idxb, body, 0)
    plsc.subcore_barrier()
    pltpu.sync_copy(deg_sh.at[stripe], out_deg.at[c, stripe])


@jax.jit
def _sc_degree(dstcat, zrows, ones_in):
    n_blocks, idxb = dstcat.shape[2], dstcat.shape[3]
    NP = zrows.shape[0]
    rows_out = NP // NS

    scratch = (
        pltpu.VMEM_SHARED((NP, 128), jnp.float32),
        pltpu.VMEM((idxb, C), jnp.int32),
        pltpu.VMEM((C, 128), jnp.float32),
    )
    mesh = plsc.VectorSubcoreMesh(core_axis_name="c", subcore_axis_name="s")
    body = functools.partial(_sc_deg_body, NP, n_blocks, idxb, rows_out)
    fn = pl.kernel(body,
                   out_type=jax.ShapeDtypeStruct((NC, NP, 128), jnp.float32),
                   mesh=mesh, scratch_types=scratch)
    return fn(dstcat, zrows, ones_in)


def _dot(a, b):
    return jnp.dot(a, b, preferred_element_type=jnp.float32, precision=_PREC)


def _tc1_body(agg_ref, degp_ref, x_ref, w1l_ref, w1r_ref, b1_ref, out_ref):
    deg = degp_ref[0, :, 0:1] + degp_ref[1, :, 0:1]
    inv = 1.0 / jnp.maximum(deg, 1.0)
    t = _dot(agg_ref[0] + agg_ref[1], w1l_ref[...])
    t = t * inv + b1_ref[...] + _dot(x_ref[...], w1r_ref[...])
    h = jnp.maximum(t, 0.0)
    out_ref[0] = h[:, :128]
    out_ref[1] = h[:, 128:]


def _tc2_body(agg_ref, degp_ref, h_ref, wll_ref, wlh_ref, wrl_ref, wrh_ref,
              b2_ref, wc_ref, bc_ref, out_ref):
    deg = degp_ref[0, :, 0:1] + degp_ref[1, :, 0:1]
    inv = 1.0 / jnp.maximum(deg, 1.0)
    t = _dot(agg_ref[0], wll_ref[...]) + _dot(agg_ref[1], wlh_ref[...])
    t = t * inv + b2_ref[...] + _dot(h_ref[0], wrl_ref[...]) \
        + _dot(h_ref[1], wrh_ref[...])
    h2 = jnp.maximum(t, 0.0)
    out_ref[...] = jnp.sum(h2 * wc_ref[...], axis=1, keepdims=True) \
        + bc_ref[...]


def _tc_layer1(agg1, degp, x, w1l, w1r, b1r, block):
    n = x.shape[0]
    grid = (n // block,)
    full = lambda i: (0, 0)
    return pl.pallas_call(
        _tc1_body,
        grid=grid,
        in_specs=[
            pl.BlockSpec((NC, block, 128), lambda i: (0, i, 0)),
            pl.BlockSpec((NC, block, 128), lambda i: (0, i, 0)),
            pl.BlockSpec((block, 128), lambda i: (i, 0)),
            pl.BlockSpec((128, 256), full),
            pl.BlockSpec((128, 256), full),
            pl.BlockSpec((1, 256), full),
        ],
        out_specs=pl.BlockSpec((NC, block, 128), lambda i: (0, i, 0)),
        out_shape=jax.ShapeDtypeStruct((NC, n, 128), jnp.float32),
    )(agg1, degp, x, w1l, w1r, b1r)


def _tc_layer2(agg2, degp, hcat, wll, wlh, wrl, wrh, b2r, wcr, bcr, block):
    n = hcat.shape[1]
    grid = (n // block,)
    full = lambda i: (0, 0)
    return pl.pallas_call(
        _tc2_body,
        grid=grid,
        in_specs=[
            pl.BlockSpec((NC, block, 128), lambda i: (0, i, 0)),
            pl.BlockSpec((NC, block, 128), lambda i: (0, i, 0)),
            pl.BlockSpec((NC, block, 128), lambda i: (0, i, 0)),
            pl.BlockSpec((128, 256), full),
            pl.BlockSpec((128, 256), full),
            pl.BlockSpec((128, 256), full),
            pl.BlockSpec((128, 256), full),
            pl.BlockSpec((1, 256), full),
            pl.BlockSpec((1, 256), full),
            pl.BlockSpec((1, 1), full),
        ],
        out_specs=pl.BlockSpec((block, 1), lambda i: (i, 0)),
        out_shape=jax.ShapeDtypeStruct((n, 1), jnp.float32),
    )(agg2, degp, hcat, wll, wlh, wrl, wrh, b2r, wcr, bcr)


def kernel(x, edge_index, W1_l, W1_r, b1, W2_l, W2_r, b2, Wc, bc):
    N, D = x.shape
    E = edge_index.shape[1]

    src = edge_index[0]
    dst = edge_index[1]
    IDXB = 25
    src1 = src.reshape(NC, NS, 5, IDXB, C)
    dst1 = dst.reshape(NC, NS, 5, IDXB, C)
    nb2 = E // C // NS // IDXB
    src2r = src.reshape(NS, nb2, IDXB, C)
    src2 = jnp.stack([src2r, src2r + N])
    dst2 = jnp.broadcast_to(dst.reshape(NS, nb2, IDXB, C),
                            (NC, NS, nb2, IDXB, C))

    NP = ((N + 8 * NS - 1) // (8 * NS)) * (8 * NS)
    z0 = jnp.zeros((NP, 128), jnp.float32)
    ones_in = jnp.ones((C, 128), jnp.float32)

    degp = _sc_degree(dst1, z0, ones_in)
    z128 = 0.0 * degp[0]

    agg1 = _sc_aggregate(x, src1, dst1, z128)

    hcat = _tc_layer1(agg1, degp, x, W1_l, W1_r, b1.reshape(1, -1),
                      block=1000)

    h2 = hcat.reshape(2 * N, 128)
    agg2 = _sc_aggregate(h2, src2, dst2, z128)

    out = _tc_layer2(agg2, degp, hcat,
                     W2_l[:128], W2_l[128:], W2_r[:128], W2_r[128:],
                     b2.reshape(1, -1), Wc.reshape(1, -1),
                     bc.reshape(1, 1), block=1000)
    return out.reshape(N)


# speedup vs baseline: 6.9012x; 1.0053x over previous
import functools

import jax
import jax.numpy as jnp
from jax import lax
from jax.experimental import pallas as pl
from jax.experimental.pallas import tpu as pltpu
from jax.experimental.pallas import tpu_sc as plsc

C = 80
NS = 16
NC = 2
_PREC = jax.lax.Precision.HIGHEST


def _sc_agg_body(NP, n_blocks, idxb, rows_out, table, srccat, dstcat, zrows,
                 out_agg, acc_sh, srcv, dstv, rows_a, rows_b,
                 sem_ga, sem_gb, sem_sa, sem_sb):
    c = lax.axis_index("c")
    s = lax.axis_index("s")

    stripe = pl.ds(s * rows_out, rows_out)
    pltpu.sync_copy(zrows.at[stripe], acc_sh.at[stripe])
    plsc.subcore_barrier()

    for b in range(n_blocks):
        pltpu.sync_copy(srccat.at[c, s, b], srcv)
        pltpu.sync_copy(dstcat.at[c, s, b], dstv)

        pltpu.async_copy(table.at[srcv.at[0]], rows_a, sem_ga)
        pltpu.async_copy(table.at[srcv.at[1]], rows_b, sem_gb)

        def body(i, carry):
            ja = 2 * i
            jb = 2 * i + 1
            pltpu.make_async_copy(table.at[srcv.at[ja]], rows_a,
                                  sem_ga).wait()
            pltpu.async_copy(rows_a, acc_sh.at[dstv.at[ja]], sem_sa,
                             add=True)
            pltpu.make_async_copy(table.at[srcv.at[jb]], rows_b,
                                  sem_gb).wait()
            pltpu.async_copy(rows_b, acc_sh.at[dstv.at[jb]], sem_sb,
                             add=True)

            @pl.when(ja + 2 < idxb)
            def _():
                pltpu.make_async_copy(rows_a, acc_sh.at[dstv.at[ja]],
                                      sem_sa).wait()
                pltpu.async_copy(table.at[srcv.at[ja + 2]], rows_a, sem_ga)

            @pl.when(jb + 2 < idxb)
            def _():
                pltpu.make_async_copy(rows_b, acc_sh.at[dstv.at[jb]],
                                      sem_sb).wait()
                pltpu.async_copy(table.at[srcv.at[jb + 2]], rows_b, sem_gb)

            return carry

        lax.fori_loop(0, idxb // 2, body, 0)
        if idxb % 2:
            pltpu.make_async_copy(table.at[srcv.at[idxb - 1]], rows_a,
                                  sem_ga).wait()
            pltpu.async_copy(rows_a, acc_sh.at[dstv.at[idxb - 1]], sem_sa,
                             add=True)
            pltpu.make_async_copy(rows_b, acc_sh.at[dstv.at[idxb - 2]],
                                  sem_sb).wait()
            pltpu.make_async_copy(rows_a, acc_sh.at[dstv.at[idxb - 1]],
                                  sem_sa).wait()
        else:
            nlast = idxb - 2
            pltpu.make_async_copy(rows_a, acc_sh.at[dstv.at[nlast]],
                                  sem_sa).wait()
            pltpu.make_async_copy(rows_b, acc_sh.at[dstv.at[nlast + 1]],
                                  sem_sb).wait()
    plsc.subcore_barrier()

    pltpu.sync_copy(acc_sh.at[stripe], out_agg.at[c, stripe])


@jax.jit
def _sc_aggregate(table, srccat, dstcat, zrows):
    n_blocks, idxb = dstcat.shape[2], dstcat.shape[3]
    NP = zrows.shape[0]
    rows_out = NP // NS

    scratch = (
        pltpu.VMEM_SHARED((NP, 128), jnp.float32),
        pltpu.VMEM((idxb, C), jnp.int32),
        pltpu.VMEM((idxb, C), jnp.int32),
        pltpu.VMEM((C, 128), jnp.float32),
        pltpu.VMEM((C, 128), jnp.float32),
        pltpu.SemaphoreType.DMA,
        pltpu.SemaphoreType.DMA,
        pltpu.SemaphoreType.DMA,
        pltpu.SemaphoreType.DMA,
    )
    mesh = plsc.VectorSubcoreMesh(core_axis_name="c", subcore_axis_name="s")
    body = functools.partial(_sc_agg_body, NP, n_blocks, idxb, rows_out)
    fn = pl.kernel(body,
                   out_type=jax.ShapeDtypeStruct((NC, NP, 128), jnp.float32),
                   mesh=mesh, scratch_types=scratch)
    return fn(table, srccat, dstcat, zrows)


def _sc_deg_body(NP, n_blocks, idxb, rows_out, dstcat, zrows, ones_in,
                 out_deg, deg_sh, dstv, ones_v):
    c = lax.axis_index("c")
    s = lax.axis_index("s")

    stripe = pl.ds(s * rows_out, rows_out)
    pltpu.sync_copy(zrows.at[stripe], deg_sh.at[stripe])
    pltpu.sync_copy(ones_in, ones_v)
    plsc.subcore_barrier()

    for b in range(n_blocks):
        pltpu.sync_copy(dstcat.at[c, s, b], dstv)

        def body(j, carry):
            pltpu.sync_copy(ones_v, deg_sh.at[dstv.at[j]], add=True)
            return carry

        lax.fori_loop(0, idxb, body, 0)
    plsc.subcore_barrier()
    pltpu.sync_copy(deg_sh.at[stripe], out_deg.at[c, stripe])


@jax.jit
def _sc_degree(dstcat, zrows, ones_in):
    n_blocks, idxb = dstcat.shape[2], dstcat.shape[3]
    NP = zrows.shape[0]
    rows_out = NP // NS

    scratch = (
        pltpu.VMEM_SHARED((NP, 128), jnp.float32),
        pltpu.VMEM((idxb, C), jnp.int32),
        pltpu.VMEM((C, 128), jnp.float32),
    )
    mesh = plsc.VectorSubcoreMesh(core_axis_name="c", subcore_axis_name="s")
    body = functools.partial(_sc_deg_body, NP, n_blocks, idxb, rows_out)
    fn = pl.kernel(body,
                   out_type=jax.ShapeDtypeStruct((NC, NP, 128), jnp.float32),
                   mesh=mesh, scratch_types=scratch)
    return fn(dstcat, zrows, ones_in)


def _dot(a, b):
    return jnp.dot(a, b, preferred_element_type=jnp.float32, precision=_PREC)


def _tc1_body(agg_ref, degp_ref, x_ref, w1l_ref, w1r_ref, b1_ref, out_ref):
    deg = degp_ref[0, :, 0:1] + degp_ref[1, :, 0:1]
    inv = 1.0 / jnp.maximum(deg, 1.0)
    t = _dot(agg_ref[0] + agg_ref[1], w1l_ref[...])
    t = t * inv + b1_ref[...] + _dot(x_ref[...], w1r_ref[...])
    h = jnp.maximum(t, 0.0)
    out_ref[0] = h[:, :128]
    out_ref[1] = h[:, 128:]


def _tc2_body(agg_ref, degp_ref, h_ref, wll_ref, wlh_ref, wrl_ref, wrh_ref,
              b2_ref, wc_ref, bc_ref, out_ref):
    deg = degp_ref[0, :, 0:1] + degp_ref[1, :, 0:1]
    inv = 1.0 / jnp.maximum(deg, 1.0)
    t = _dot(agg_ref[0], wll_ref[...]) + _dot(agg_ref[1], wlh_ref[...])
    t = t * inv + b2_ref[...] + _dot(h_ref[0], wrl_ref[...]) \
        + _dot(h_ref[1], wrh_ref[...])
    h2 = jnp.maximum(t, 0.0)
    out_ref[...] = jnp.sum(h2 * wc_ref[...], axis=1, keepdims=True) \
        + bc_ref[...]


def _tc_layer1(agg1, degp, x, w1l, w1r, b1r, block):
    n = x.shape[0]
    grid = (n // block,)
    full = lambda i: (0, 0)
    return pl.pallas_call(
        _tc1_body,
        grid=grid,
        in_specs=[
            pl.BlockSpec((NC, block, 128), lambda i: (0, i, 0)),
            pl.BlockSpec((NC, block, 128), lambda i: (0, i, 0)),
            pl.BlockSpec((block, 128), lambda i: (i, 0)),
            pl.BlockSpec((128, 256), full),
            pl.BlockSpec((128, 256), full),
            pl.BlockSpec((1, 256), full),
        ],
        out_specs=pl.BlockSpec((NC, block, 128), lambda i: (0, i, 0)),
        out_shape=jax.ShapeDtypeStruct((NC, n, 128), jnp.float32),
    )(agg1, degp, x, w1l, w1r, b1r)


def _tc_layer2(agg2, degp, hcat, wll, wlh, wrl, wrh, b2r, wcr, bcr, block):
    n = hcat.shape[1]
    grid = (n // block,)
    full = lambda i: (0, 0)
    return pl.pallas_call(
        _tc2_body,
        grid=grid,
        in_specs=[
            pl.BlockSpec((NC, block, 128), lambda i: (0, i, 0)),
            pl.BlockSpec((NC, block, 128), lambda i: (0, i, 0)),
            pl.BlockSpec((NC, block, 128), lambda i: (0, i, 0)),
            pl.BlockSpec((128, 256), full),
            pl.BlockSpec((128, 256), full),
            pl.BlockSpec((128, 256), full),
            pl.BlockSpec((128, 256), full),
            pl.BlockSpec((1, 256), full),
            pl.BlockSpec((1, 256), full),
            pl.BlockSpec((1, 1), full),
        ],
        out_specs=pl.BlockSpec((block, 1), lambda i: (i, 0)),
        out_shape=jax.ShapeDtypeStruct((n, 1), jnp.float32),
    )(agg2, degp, hcat, wll, wlh, wrl, wrh, b2r, wcr, bcr)


def kernel(x, edge_index, W1_l, W1_r, b1, W2_l, W2_r, b2, Wc, bc):
    N, D = x.shape
    E = edge_index.shape[1]

    src = edge_index[0]
    dst = edge_index[1]
    IDXB = 25
    src1 = src.reshape(NC, NS, 5, IDXB, C)
    dst1 = dst.reshape(NC, NS, 5, IDXB, C)
    nb2 = E // C // NS // IDXB
    src2r = src.reshape(NS, nb2, IDXB, C)
    src2 = jnp.stack([src2r, src2r + N])
    dst2 = jnp.broadcast_to(dst.reshape(NS, nb2, IDXB, C),
                            (NC, NS, nb2, IDXB, C))

    NP = ((N + 8 * NS - 1) // (8 * NS)) * (8 * NS)
    z0 = jnp.zeros((NP, 128), jnp.float32)
    ones_in = jnp.ones((C, 128), jnp.float32)

    degp = _sc_degree(dst1, z0, ones_in)
    z128 = 0.0 * degp[0]

    agg1 = _sc_aggregate(x, src1, dst1, z128)

    hcat = _tc_layer1(agg1, degp, x, W1_l, W1_r, b1.reshape(1, -1),
                      block=1000)

    h2 = hcat.reshape(2 * N, 128)
    agg2 = _sc_aggregate(h2, src2, dst2, z128)

    out = _tc_layer2(agg2, degp, hcat,
                     W2_l[:128], W2_l[128:], W2_r[:128], W2_r[128:],
                     b2.reshape(1, -1), Wc.reshape(1, -1),
                     bc.reshape(1, 1), block=1000)
    return out.reshape(N)


# speedup vs baseline: 7.2796x; 1.0548x over previous
import functools

import jax
import jax.numpy as jnp
from jax import lax
from jax.experimental import pallas as pl
from jax.experimental.pallas import tpu as pltpu
from jax.experimental.pallas import tpu_sc as plsc

C = 125
NS = 16
NC = 2
_PREC = jax.lax.Precision.HIGHEST


def _sc_agg_body(NP, n_blocks, idxb, rows_out, table, srccat, dstcat, zrows,
                 out_agg, acc_sh, srcv, dstv, rows_a, rows_b,
                 sem_ga, sem_gb, sem_sa, sem_sb):
    c = lax.axis_index("c")
    s = lax.axis_index("s")

    stripe = pl.ds(s * rows_out, rows_out)
    pltpu.sync_copy(zrows.at[stripe], acc_sh.at[stripe])
    plsc.subcore_barrier()

    for b in range(n_blocks):
        pltpu.sync_copy(srccat.at[c, s, b], srcv)
        pltpu.sync_copy(dstcat.at[c, s, b], dstv)

        pltpu.async_copy(table.at[srcv.at[0]], rows_a, sem_ga)
        pltpu.async_copy(table.at[srcv.at[1]], rows_b, sem_gb)

        def body(i, carry):
            ja = 2 * i
            jb = 2 * i + 1
            pltpu.make_async_copy(table.at[srcv.at[ja]], rows_a,
                                  sem_ga).wait()
            pltpu.async_copy(rows_a, acc_sh.at[dstv.at[ja]], sem_sa,
                             add=True)
            pltpu.make_async_copy(table.at[srcv.at[jb]], rows_b,
                                  sem_gb).wait()
            pltpu.async_copy(rows_b, acc_sh.at[dstv.at[jb]], sem_sb,
                             add=True)

            @pl.when(ja + 2 < idxb)
            def _():
                pltpu.make_async_copy(rows_a, acc_sh.at[dstv.at[ja]],
                                      sem_sa).wait()
                pltpu.async_copy(table.at[srcv.at[ja + 2]], rows_a, sem_ga)

            @pl.when(jb + 2 < idxb)
            def _():
                pltpu.make_async_copy(rows_b, acc_sh.at[dstv.at[jb]],
                                      sem_sb).wait()
                pltpu.async_copy(table.at[srcv.at[jb + 2]], rows_b, sem_gb)

            return carry

        lax.fori_loop(0, idxb // 2, body, 0)
        if idxb % 2:
            pltpu.make_async_copy(table.at[srcv.at[idxb - 1]], rows_a,
                                  sem_ga).wait()
            pltpu.async_copy(rows_a, acc_sh.at[dstv.at[idxb - 1]], sem_sa,
                             add=True)
            pltpu.make_async_copy(rows_b, acc_sh.at[dstv.at[idxb - 2]],
                                  sem_sb).wait()
            pltpu.make_async_copy(rows_a, acc_sh.at[dstv.at[idxb - 1]],
                                  sem_sa).wait()
        else:
            nlast = idxb - 2
            pltpu.make_async_copy(rows_a, acc_sh.at[dstv.at[nlast]],
                                  sem_sa).wait()
            pltpu.make_async_copy(rows_b, acc_sh.at[dstv.at[nlast + 1]],
                                  sem_sb).wait()
    plsc.subcore_barrier()

    pltpu.sync_copy(acc_sh.at[stripe], out_agg.at[c, stripe])


@jax.jit
def _sc_aggregate(table, srccat, dstcat, zrows):
    n_blocks, idxb = dstcat.shape[2], dstcat.shape[3]
    NP = zrows.shape[0]
    rows_out = NP // NS

    scratch = (
        pltpu.VMEM_SHARED((NP, 128), jnp.float32),
        pltpu.VMEM((idxb, C), jnp.int32),
        pltpu.VMEM((idxb, C), jnp.int32),
        pltpu.VMEM((C, 128), jnp.float32),
        pltpu.VMEM((C, 128), jnp.float32),
        pltpu.SemaphoreType.DMA,
        pltpu.SemaphoreType.DMA,
        pltpu.SemaphoreType.DMA,
        pltpu.SemaphoreType.DMA,
    )
    mesh = plsc.VectorSubcoreMesh(core_axis_name="c", subcore_axis_name="s")
    body = functools.partial(_sc_agg_body, NP, n_blocks, idxb, rows_out)
    fn = pl.kernel(body,
                   out_type=jax.ShapeDtypeStruct((NC, NP, 128), jnp.float32),
                   mesh=mesh, scratch_types=scratch)
    return fn(table, srccat, dstcat, zrows)


def _sc_deg_body(NP, n_blocks, idxb, rows_out, dstcat, zrows, ones_in,
                 out_deg, deg_sh, dstv, ones_v):
    c = lax.axis_index("c")
    s = lax.axis_index("s")

    stripe = pl.ds(s * rows_out, rows_out)
    pltpu.sync_copy(zrows.at[stripe], deg_sh.at[stripe])
    pltpu.sync_copy(ones_in, ones_v)
    plsc.subcore_barrier()

    for b in range(n_blocks):
        pltpu.sync_copy(dstcat.at[c, s, b], dstv)

        def body(j, carry):
            pltpu.sync_copy(ones_v, deg_sh.at[dstv.at[j]], add=True)
            return carry

        lax.fori_loop(0, idxb, body, 0)
    plsc.subcore_barrier()
    pltpu.sync_copy(deg_sh.at[stripe], out_deg.at[c, stripe])


@jax.jit
def _sc_degree(dstcat, zrows, ones_in):
    n_blocks, idxb = dstcat.shape[2], dstcat.shape[3]
    NP = zrows.shape[0]
    rows_out = NP // NS

    scratch = (
        pltpu.VMEM_SHARED((NP, 128), jnp.float32),
        pltpu.VMEM((idxb, C), jnp.int32),
        pltpu.VMEM((C, 128), jnp.float32),
    )
    mesh = plsc.VectorSubcoreMesh(core_axis_name="c", subcore_axis_name="s")
    body = functools.partial(_sc_deg_body, NP, n_blocks, idxb, rows_out)
    fn = pl.kernel(body,
                   out_type=jax.ShapeDtypeStruct((NC, NP, 128), jnp.float32),
                   mesh=mesh, scratch_types=scratch)
    return fn(dstcat, zrows, ones_in)


def _dot(a, b):
    return jnp.dot(a, b, preferred_element_type=jnp.float32, precision=_PREC)


def _tc1_body(agg_ref, degp_ref, x_ref, w1l_ref, w1r_ref, b1_ref, out_ref):
    deg = degp_ref[0, :, 0:1] + degp_ref[1, :, 0:1]
    inv = 1.0 / jnp.maximum(deg, 1.0)
    t = _dot(agg_ref[0] + agg_ref[1], w1l_ref[...])
    t = t * inv + b1_ref[...] + _dot(x_ref[...], w1r_ref[...])
    h = jnp.maximum(t, 0.0)
    out_ref[0] = h[:, :128]
    out_ref[1] = h[:, 128:]


def _tc2_body(agg_ref, degp_ref, h_ref, wll_ref, wlh_ref, wrl_ref, wrh_ref,
              b2_ref, wc_ref, bc_ref, out_ref):
    deg = degp_ref[0, :, 0:1] + degp_ref[1, :, 0:1]
    inv = 1.0 / jnp.maximum(deg, 1.0)
    t = _dot(agg_ref[0], wll_ref[...]) + _dot(agg_ref[1], wlh_ref[...])
    t = t * inv + b2_ref[...] + _dot(h_ref[0], wrl_ref[...]) \
        + _dot(h_ref[1], wrh_ref[...])
    h2 = jnp.maximum(t, 0.0)
    out_ref[...] = jnp.sum(h2 * wc_ref[...], axis=1, keepdims=True) \
        + bc_ref[...]


def _tc_layer1(agg1, degp, x, w1l, w1r, b1r, block):
    n = x.shape[0]
    grid = (n // block,)
    full = lambda i: (0, 0)
    return pl.pallas_call(
        _tc1_body,
        grid=grid,
        in_specs=[
            pl.BlockSpec((NC, block, 128), lambda i: (0, i, 0)),
            pl.BlockSpec((NC, block, 128), lambda i: (0, i, 0)),
            pl.BlockSpec((block, 128), lambda i: (i, 0)),
            pl.BlockSpec((128, 256), full),
            pl.BlockSpec((128, 256), full),
            pl.BlockSpec((1, 256), full),
        ],
        out_specs=pl.BlockSpec((NC, block, 128), lambda i: (0, i, 0)),
        out_shape=jax.ShapeDtypeStruct((NC, n, 128), jnp.float32),
    )(agg1, degp, x, w1l, w1r, b1r)


def _tc_layer2(agg2, degp, hcat, wll, wlh, wrl, wrh, b2r, wcr, bcr, block):
    n = hcat.shape[1]
    grid = (n // block,)
    full = lambda i: (0, 0)
    return pl.pallas_call(
        _tc2_body,
        grid=grid,
        in_specs=[
            pl.BlockSpec((NC, block, 128), lambda i: (0, i, 0)),
            pl.BlockSpec((NC, block, 128), lambda i: (0, i, 0)),
            pl.BlockSpec((NC, block, 128), lambda i: (0, i, 0)),
            pl.BlockSpec((128, 256), full),
            pl.BlockSpec((128, 256), full),
            pl.BlockSpec((128, 256), full),
            pl.BlockSpec((128, 256), full),
            pl.BlockSpec((1, 256), full),
            pl.BlockSpec((1, 256), full),
            pl.BlockSpec((1, 1), full),
        ],
        out_specs=pl.BlockSpec((block, 1), lambda i: (i, 0)),
        out_shape=jax.ShapeDtypeStruct((n, 1), jnp.float32),
    )(agg2, degp, hcat, wll, wlh, wrl, wrh, b2r, wcr, bcr)


def kernel(x, edge_index, W1_l, W1_r, b1, W2_l, W2_r, b2, Wc, bc):
    N, D = x.shape
    E = edge_index.shape[1]

    src = edge_index[0]
    dst = edge_index[1]
    IDXB = 16
    nb1 = E // C // (NC * NS) // IDXB
    src1 = src.reshape(NC, NS, nb1, IDXB, C)
    dst1 = dst.reshape(NC, NS, nb1, IDXB, C)
    nb2 = E // C // NS // IDXB
    src2r = src.reshape(NS, nb2, IDXB, C)
    src2 = jnp.stack([src2r, src2r + N])
    dst2 = jnp.broadcast_to(dst.reshape(NS, nb2, IDXB, C),
                            (NC, NS, nb2, IDXB, C))

    NP = ((N + 8 * NS - 1) // (8 * NS)) * (8 * NS)
    z0 = jnp.zeros((NP, 128), jnp.float32)
    ones_in = jnp.ones((C, 128), jnp.float32)

    degp = _sc_degree(dst1, z0, ones_in)
    z128 = 0.0 * degp[0]

    agg1 = _sc_aggregate(x, src1, dst1, z128)

    hcat = _tc_layer1(agg1, degp, x, W1_l, W1_r, b1.reshape(1, -1),
                      block=1000)

    h2 = hcat.reshape(2 * N, 128)
    agg2 = _sc_aggregate(h2, src2, dst2, z128)

    out = _tc_layer2(agg2, degp, hcat,
                     W2_l[:128], W2_l[128:], W2_r[:128], W2_r[128:],
                     b2.reshape(1, -1), Wc.reshape(1, -1),
                     bc.reshape(1, 1), block=1000)
    return out.reshape(N)
